# baseline (speedup 1.0000x reference)
import functools

import jax
import jax.numpy as jnp
from jax import lax
from jax.experimental import pallas as pl
from jax.experimental.pallas import tpu as pltpu
from jax.experimental.pallas import tpu_sc as plsc

B = 16384
IN_DIM = 100
N_CAT = 26
VOCAB = 100000
EMB = 3
CAT_W = N_CAT * EMB
CONT_W = IN_DIM - N_CAT

NC, NS = 2, 16
NW = NC * NS
R = B // NW
C = 256
NCH = R // C
QC = C * CAT_W
NIDX = QC // 128

_mesh = plsc.VectorSubcoreMesh(core_axis_name="c", subcore_axis_name="s")


@functools.partial(
    pl.kernel,
    out_type=(
        jax.ShapeDtypeStruct((NW, NCH, NIDX, 128), jnp.float32),
        jax.ShapeDtypeStruct((B, IN_DIM), jnp.float32),
    ),
    mesh=_mesh,
    compiler_params=pltpu.CompilerParams(
        use_tc_tiling_on_sc=False, needs_layout_passes=False),
    scratch_types=[
        pltpu.VMEM((C, IN_DIM), jnp.float32),
        pltpu.VMEM((NIDX, 128), jnp.int32),
        pltpu.VMEM((NIDX, 128), jnp.float32),
        pltpu.SemaphoreType.DMA,
        pltpu.SemaphoreType.DMA,
        pltpu.SemaphoreType.DMA,
    ],
)
def _emb_kernel(xb_hbm, tbl_hbm, cat_hbm, xf_hbm, xv, idxv, gv,
                lsem, gsem, osem):
    wid = lax.axis_index("s") * NC + lax.axis_index("c")
    lane = lax.iota(jnp.int32, 16)
    tail_cols = jnp.minimum(lane + 90, IN_DIM - 1)

    for ch in range(NCH):
        base = wid * R + ch * C
        pltpu.async_copy(xb_hbm.at[pl.ds(base, C), :], xv, lsem).wait()

        @pl.loop(0, NIDX)
        def _build(v):
            for u in range(8):
                q = lane + (v * 128 + u * 16)
                m = lax.shift_right_logical(q * 43691, 17)
                k = q - m * 3
                r = lax.shift_right_logical(m * 20165, 19)
                j = m - r * N_CAT
                bits = plsc.load_gather(xv, [r, j])
                xi = plsc.bitcast(bits, jnp.int32)
                idxv[v, pl.ds(u * 16, 16)] = (
                    xi * EMB + (j * (VOCAB * EMB) + k))

        copies = [
            pltpu.async_copy(tbl_hbm.at[idxv.at[v]], gv.at[v], gsem)
            for v in range(NIDX)
        ]

        @pl.loop(0, C)
        def _convert(r):
            for c in (N_CAT, N_CAT + 16, N_CAT + 32, N_CAT + 48):
                bits = xv[r, pl.ds(c, 16)]
                xv[r, pl.ds(c, 16)] = (
                    plsc.bitcast(bits, jnp.int32).astype(jnp.float32))
            rows = jnp.full((16,), r, dtype=jnp.int32)
            bits = plsc.load_gather(xv, [rows, tail_cols])
            vals = plsc.bitcast(bits, jnp.int32).astype(jnp.float32)
            plsc.store_scatter(xv, [rows, tail_cols], vals)

        for cp in copies:
            cp.wait()

        pltpu.async_copy(gv, cat_hbm.at[wid, ch], osem).wait()
        pltpu.async_copy(xv, xf_hbm.at[pl.ds(base, C), :], osem).wait()


def kernel(x, tables):
    xb = lax.bitcast_convert_type(x, jnp.float32)
    tbl = tables.reshape(N_CAT * VOCAB * EMB)
    cat, xf = _emb_kernel(xb, tbl)
    return jnp.concatenate(
        [cat.reshape(B, CAT_W), xf[:, N_CAT:]], axis=1)


# speedup vs baseline: 15.9851x; 15.9851x over previous
import functools

import jax
import jax.numpy as jnp
from jax import lax
from jax.experimental import pallas as pl
from jax.experimental.pallas import tpu as pltpu
from jax.experimental.pallas import tpu_sc as plsc

B = 16384
IN_DIM = 100
N_CAT = 26
VOCAB = 100000
EMB = 3
CAT_W = N_CAT * EMB
CONT_W = IN_DIM - N_CAT

NC, NS = 2, 16
NW = NC * NS
R = B // NW
C = 256
NCH = R // C
QC = C * CAT_W
NIDX = QC // 128

_mesh = plsc.VectorSubcoreMesh(core_axis_name="c", subcore_axis_name="s")


@functools.partial(
    pl.kernel,
    out_type=(
        jax.ShapeDtypeStruct((NW, NCH, NIDX, 128), jnp.float32),
        jax.ShapeDtypeStruct((B, IN_DIM), jnp.float32),
    ),
    mesh=_mesh,
    compiler_params=pltpu.CompilerParams(
        use_tc_tiling_on_sc=False, needs_layout_passes=False),
    scratch_types=[
        pltpu.VMEM((C, IN_DIM), jnp.float32),
        pltpu.VMEM((NIDX, 128), jnp.int32),
        pltpu.VMEM((NIDX, 128), jnp.float32),
        pltpu.SemaphoreType.DMA,
        pltpu.SemaphoreType.DMA,
        pltpu.SemaphoreType.DMA,
    ],
)
def _emb_kernel(xb_hbm, tbl_hbm, cat_hbm, xf_hbm, xv, idxv, gv,
                lsem, gsem, osem):
    wid = lax.axis_index("s") * NC + lax.axis_index("c")
    lane = lax.iota(jnp.int32, 16)
    tail_cols = jnp.minimum(lane + 90, IN_DIM - 1)

    for ch in range(NCH):
        base = wid * R + ch * C
        pltpu.async_copy(xb_hbm.at[pl.ds(base, C), :], xv, lsem).wait()

        @pl.loop(0, NIDX)
        def _build(v):
            for u in range(8):
                q = lane + (v * 128 + u * 16)
                m = lax.shift_right_logical(q * 43691, 17)
                k = q - m * 3
                r = lax.shift_right_logical(m * 20165, 19)
                j = m - r * N_CAT
                bits = plsc.load_gather(xv, [r, j])
                xi = plsc.bitcast(bits, jnp.int32)
                idxv[v, pl.ds(u * 16, 16)] = (
                    xi + (k * (N_CAT * VOCAB) + j * VOCAB))

        copies = [
            pltpu.async_copy(tbl_hbm.at[idxv.at[v]], gv.at[v], gsem)
            for v in range(NIDX)
        ]

        @pl.loop(0, C)
        def _convert(r):
            for c in (N_CAT, N_CAT + 16, N_CAT + 32, N_CAT + 48):
                bits = xv[r, pl.ds(c, 16)]
                xv[r, pl.ds(c, 16)] = (
                    plsc.bitcast(bits, jnp.int32).astype(jnp.float32))
            rows = jnp.full((16,), r, dtype=jnp.int32)
            bits = plsc.load_gather(xv, [rows, tail_cols])
            vals = plsc.bitcast(bits, jnp.int32).astype(jnp.float32)
            plsc.store_scatter(xv, [rows, tail_cols], vals)

        for cp in copies:
            cp.wait()

        pltpu.async_copy(gv, cat_hbm.at[wid, ch], osem).wait()
        pltpu.async_copy(xv, xf_hbm.at[pl.ds(base, C), :], osem).wait()


def kernel(x, tables):
    xb = lax.bitcast_convert_type(x, jnp.float32)
    tbl = tables.transpose(2, 0, 1).reshape(N_CAT * VOCAB * EMB)
    cat, xf = _emb_kernel(xb, tbl)
    return jnp.concatenate(
        [cat.reshape(B, CAT_W), xf[:, N_CAT:]], axis=1)


# speedup vs baseline: 17.3021x; 1.0824x over previous
import functools

import jax
import jax.numpy as jnp
from jax import lax
from jax.experimental import pallas as pl
from jax.experimental.pallas import tpu as pltpu
from jax.experimental.pallas import tpu_sc as plsc

B = 16384
IN_DIM = 100
N_CAT = 26
VOCAB = 100000
EMB = 3
CAT_W = N_CAT * EMB
CONT_W = IN_DIM - N_CAT

NC, NS = 2, 16
NW = NC * NS
R = B // NW
C = 256
NCH = R // C
QC = C * CAT_W
NIDX = QC // 128

_mesh = plsc.VectorSubcoreMesh(core_axis_name="c", subcore_axis_name="s")


@functools.partial(
    pl.kernel,
    out_type=(
        jax.ShapeDtypeStruct((NW, NCH, NIDX, 128), jnp.float32),
        jax.ShapeDtypeStruct((B, IN_DIM), jnp.float32),
    ),
    mesh=_mesh,
    compiler_params=pltpu.CompilerParams(
        use_tc_tiling_on_sc=False, needs_layout_passes=False),
    scratch_types=[
        pltpu.VMEM((C, IN_DIM), jnp.float32),
        pltpu.VMEM((NIDX, 128), jnp.int32),
        pltpu.VMEM((NIDX, 128), jnp.float32),
        pltpu.SemaphoreType.DMA,
        pltpu.SemaphoreType.DMA,
        pltpu.SemaphoreType.DMA,
    ],
)
def _emb_kernel(xb_hbm, tbl_hbm, cat_hbm, xf_hbm, xv, idxv, gv,
                lsem, gsem, osem):
    wid = lax.axis_index("s") * NC + lax.axis_index("c")
    lane = lax.iota(jnp.int32, 16)
    tail_cols = jnp.minimum(lane + 90, IN_DIM - 1)

    for ch in range(NCH):
        base = wid * R + ch * C
        pltpu.async_copy(xb_hbm.at[pl.ds(base, C), :], xv, lsem).wait()

        @pl.loop(0, NIDX)
        def _build(v):
            for u in range(8):
                q = lane + (v * 128 + u * 16)
                m = lax.shift_right_logical(q * 43691, 17)
                k = q - m * 3
                r = lax.shift_right_logical(m * 20165, 19)
                j = m - r * N_CAT
                bits = plsc.load_gather(xv, [r, j])
                xi = plsc.bitcast(bits, jnp.int32)
                idxv[v, pl.ds(u * 16, 16)] = (
                    xi + (k * (N_CAT * VOCAB) + j * VOCAB))

        copies = [
            pltpu.async_copy(tbl_hbm.at[idxv.at[v]], gv.at[v], gsem)
            for v in range(NIDX)
        ]

        @pl.loop(0, C)
        def _convert(r):
            for c in (N_CAT, N_CAT + 16, N_CAT + 32, N_CAT + 48):
                bits = xv[r, pl.ds(c, 16)]
                xv[r, pl.ds(c, 16)] = (
                    plsc.bitcast(bits, jnp.int32).astype(jnp.float32))
            rows = jnp.full((16,), r, dtype=jnp.int32)
            bits = plsc.load_gather(xv, [rows, tail_cols])
            vals = plsc.bitcast(bits, jnp.int32).astype(jnp.float32)
            plsc.store_scatter(xv, [rows, tail_cols], vals)

        for cp in copies:
            cp.wait()

        pltpu.async_copy(gv, cat_hbm.at[wid, ch], osem).wait()
        pltpu.async_copy(xv, xf_hbm.at[pl.ds(base, C), :], osem).wait()


def kernel(x, tables):
    xb = lax.bitcast_convert_type(x, jnp.float32)
    tbl = jnp.concatenate(
        [tables[:, :, k].reshape(N_CAT * VOCAB) for k in range(EMB)])
    cat, xf = _emb_kernel(xb, tbl)
    return jnp.concatenate(
        [cat.reshape(B, CAT_W), xf[:, N_CAT:]], axis=1)


# speedup vs baseline: 37.7428x; 2.1814x over previous
import functools

import jax
import jax.numpy as jnp
from jax import lax
from jax.experimental import pallas as pl
from jax.experimental.pallas import tpu as pltpu
from jax.experimental.pallas import tpu_sc as plsc

B = 16384
IN_DIM = 100
N_CAT = 26
VOCAB = 100000
EMB = 3
CAT_W = N_CAT * EMB
OUT_W = CAT_W + IN_DIM - N_CAT

NC, NS = 2, 16
NW = NC * NS
R = B // NW
C = 256
NCH = R // C
MC = C * N_CAT
NIDX = MC // 128

_mesh = plsc.VectorSubcoreMesh(core_axis_name="c", subcore_axis_name="s")


@functools.partial(
    pl.kernel,
    out_type=jax.ShapeDtypeStruct((B, OUT_W), jnp.float32),
    mesh=_mesh,
    compiler_params=pltpu.CompilerParams(
        use_tc_tiling_on_sc=False, needs_layout_passes=False),
    scratch_types=[
        pltpu.VMEM((C, IN_DIM), jnp.float32),
        pltpu.VMEM((NIDX, 128), jnp.int32),
        pltpu.VMEM((EMB, NIDX, 128), jnp.float32),
        pltpu.VMEM((C, OUT_W), jnp.float32),
        pltpu.SemaphoreType.DMA,
        pltpu.SemaphoreType.DMA,
        pltpu.SemaphoreType.DMA,
    ],
)
def _emb_kernel(xb_hbm, t0_hbm, t1_hbm, t2_hbm, out_hbm,
                xv, idxv, gq, ov, lsem, gsem, osem):
    wid = lax.axis_index("s") * NC + lax.axis_index("c")
    lane = lax.iota(jnp.int32, 16)

    rep = []
    for c0 in (0, 16, 32, 48, 62):
        w = lane + c0
        m3 = lax.shift_right_logical(w * 21846, 16)
        rep.append((c0, m3, w - m3 * 3))

    for ch in range(NCH):
        base = wid * R + ch * C
        pltpu.async_copy(xb_hbm.at[pl.ds(base, C), :], xv, lsem).wait()

        @pl.loop(0, NIDX)
        def _build(v):
            for u in range(8):
                m = lane + (v * 128 + u * 16)
                r = lax.shift_right_logical(m * 20165, 19)
                j = m - r * N_CAT
                bits = plsc.load_gather(xv, [r, j])
                idxv[v, pl.ds(u * 16, 16)] = (
                    plsc.bitcast(bits, jnp.int32) + j * VOCAB)

        copies = []
        for v in range(NIDX):
            iv = idxv.at[v]
            copies.append(pltpu.async_copy(t0_hbm.at[iv], gq.at[0, v], gsem))
            copies.append(pltpu.async_copy(t1_hbm.at[iv], gq.at[1, v], gsem))
            copies.append(pltpu.async_copy(t2_hbm.at[iv], gq.at[2, v], gsem))

        @pl.loop(0, C)
        def _convert(r):
            for c in (26, 42, 58, 74, 84):
                bits = xv[r, pl.ds(c, 16)]
                ov[r, pl.ds(c + 52, 16)] = (
                    plsc.bitcast(bits, jnp.int32).astype(jnp.float32))

        for cp in copies:
            cp.wait()

        @pl.loop(0, C)
        def _repack(r):
            r26 = r * N_CAT
            for c0, m3, k in rep:
                m = m3 + r26
                a = lax.shift_right_logical(m, 7)
                b = lax.bitwise_and(m, 127)
                ov[r, pl.ds(c0, 16)] = plsc.load_gather(gq, [k, a, b])

        pltpu.async_copy(ov, out_hbm.at[pl.ds(base, C), :], osem).wait()


def kernel(x, tables):
    xb = lax.bitcast_convert_type(x, jnp.float32)
    planes = [tables[:, :, k].reshape(N_CAT * VOCAB) for k in range(EMB)]
    return _emb_kernel(xb, *planes)
